# baseline (speedup 1.0000x reference)
import functools

import jax
import jax.numpy as jnp
from jax import lax
from jax.experimental import pallas as pl
from jax.experimental.pallas import tpu as pltpu
from jax.experimental.pallas import tpu_sc as plsc

N = 50000
B = 128
D = 256
BLK = 10000

NW = 32
SC_CHUNK = 1568
SC_PAD = NW * SC_CHUNK - N


@functools.partial(
    pl.kernel,
    out_type=jax.ShapeDtypeStruct((NW, 2 * B), jnp.int32),
    mesh=plsc.VectorSubcoreMesh(core_axis_name="c", subcore_axis_name="s"),
    compiler_params=pltpu.CompilerParams(needs_layout_passes=False),
    scratch_types=[
        pltpu.VMEM((SC_CHUNK + 2,), jnp.int32),
        pltpu.VMEM((2 * B,), jnp.int32),
    ],
)
def _sc_bounds(batch_hbm, out_hbm, chunk_v, fe_v):
    wid = lax.axis_index("s") * 2 + lax.axis_index("c")
    base = wid * SC_CHUNK
    pltpu.sync_copy(batch_hbm.at[pl.ds(base, SC_CHUNK + 2)], chunk_v)

    zeros16 = jnp.zeros((16,), jnp.int32)

    def zero_body(i, _):
        fe_v[pl.ds(i * 16, 16)] = zeros16
        return 0

    lax.fori_loop(0, (2 * B) // 16, zero_body, 0)

    gv0 = lax.iota(jnp.int32, 16) + (base + 1)

    def scat_body(g, _):
        cur = chunk_v[pl.ds(g * 16 + 1, 16)]
        prv = chunk_v[pl.ds(g * 16, 16)]
        nxt = chunk_v[pl.ds(g * 16 + 2, 16)]
        gp1 = gv0 + g * 16
        plsc.store_scatter(fe_v, [cur], gp1, mask=cur != prv)
        plsc.store_scatter(fe_v, [cur + B], gp1, mask=cur != nxt)
        return 0

    lax.fori_loop(0, SC_CHUNK // 16, scat_body, 0)
    pltpu.sync_copy(fe_v, out_hbm.at[wid])


def _scale_kernel(x_ref, fe_ref, out_ref):
    fe = jnp.sum(fe_ref[...], axis=0, keepdims=True)
    firsts = fe[:, :B]
    ends = fe[:, B:]
    lane = lax.broadcasted_iota(jnp.int32, (1, B), 1)
    ends = ends - jnp.where(lane == B - 1, SC_PAD, 0)
    occ = ends > 0
    starts = jnp.where(occ, firsts - 1, 0)
    ends = jnp.where(occ, ends, 0)
    deg_f = (ends - starts).astype(jnp.float32)
    inv = lax.rsqrt(jnp.maximum(deg_f, 1.0))

    i = pl.program_id(0)
    rows = lax.broadcasted_iota(jnp.int32, (BLK, B), 0) + i * BLK
    oh = ((rows >= starts) & (rows < ends)).astype(jnp.float32)
    scale = lax.dot_general(
        oh, inv, (((1,), (1,)), ((), ())),
        preferred_element_type=jnp.float32,
    )
    out_ref[...] = x_ref[...] * scale


def kernel(x, batch):
    b32 = batch.astype(jnp.int32)
    bext = jnp.concatenate([
        jnp.full((1,), -1, jnp.int32),
        b32,
        jnp.full((SC_PAD,), B - 1, jnp.int32),
        jnp.full((1,), B, jnp.int32),
    ])

    fe = _sc_bounds(bext)

    out = pl.pallas_call(
        _scale_kernel,
        grid=(N // BLK,),
        in_specs=[
            pl.BlockSpec((BLK, D), lambda i: (i, 0)),
            pl.BlockSpec((NW, 2 * B), lambda i: (0, 0)),
        ],
        out_specs=pl.BlockSpec((BLK, D), lambda i: (i, 0)),
        out_shape=jax.ShapeDtypeStruct(x.shape, x.dtype),
    )(x, fe)
    return out


# speedup vs baseline: 1.5943x; 1.5943x over previous
import jax
import jax.numpy as jnp
from jax import lax
from jax.experimental import pallas as pl
from jax.experimental.pallas import tpu as pltpu

N = 50000
B = 128
D = 256
BLK = 10000
PAD127 = 48
ROWS = (N + PAD127) // 128 + 1


def _fused_kernel(x_ref, bfull_ref, out_ref, se_ref, inv_ref):
    @pl.when(pl.program_id(0) == 0)
    def _compute_bounds():
        A = bfull_ref[...]
        lane = lax.broadcasted_iota(jnp.int32, (1, B), 1)
        rowmax = jnp.max(A, axis=1, keepdims=True)
        full = jnp.sum((rowmax <= lane).astype(jnp.float32), axis=0,
                       keepdims=True)
        eye = (
            lax.broadcasted_iota(jnp.int32, (B, B), 0)
            == lax.broadcasted_iota(jnp.int32, (B, B), 1)
        ).astype(jnp.float32)
        full_t = lax.dot_general(
            eye, full, (((1,), (1,)), ((), ())),
            preferred_element_type=jnp.float32,
        )
        rowid = lax.broadcasted_iota(jnp.int32, (B, ROWS), 1).astype(
            jnp.float32)
        ohsel = (rowid == full_t).astype(jnp.float32)
        selrow = lax.dot_general(
            ohsel, A.astype(jnp.float32), (((1,), (0,)), ((), ())),
            preferred_element_type=jnp.float32,
        )
        bsub = lax.broadcasted_iota(jnp.int32, (B, 1), 0).astype(
            jnp.float32)
        part = jnp.sum((selrow <= bsub).astype(jnp.float32), axis=1,
                       keepdims=True)
        cnt_sub = full_t * 128.0 + part
        ends_f = lax.dot_general(
            cnt_sub, eye, (((0,), (0,)), ((), ())),
            preferred_element_type=jnp.float32,
        )
        shift = (
            lax.broadcasted_iota(jnp.int32, (B, B), 0)
            == lax.broadcasted_iota(jnp.int32, (B, B), 1) - 1
        ).astype(jnp.float32)
        starts_f = lax.dot_general(
            cnt_sub, shift, (((0,), (0,)), ((), ())),
            preferred_element_type=jnp.float32,
        )
        ends = ends_f.astype(jnp.int32)
        ends = jnp.where(lane == B - 1, N, ends)
        starts = starts_f.astype(jnp.int32)
        deg_f = (ends - starts).astype(jnp.float32)
        se_ref[0:1, :] = starts
        se_ref[1:2, :] = ends
        inv_ref[...] = lax.rsqrt(jnp.maximum(deg_f, 1.0))

    i = pl.program_id(0)
    rows = lax.broadcasted_iota(jnp.int32, (BLK, B), 0) + i * BLK
    oh = (
        (rows >= se_ref[0:1, :]) & (rows < se_ref[1:2, :])
    ).astype(jnp.float32)
    scale = lax.dot_general(
        oh, inv_ref[...], (((1,), (1,)), ((), ())),
        preferred_element_type=jnp.float32,
    )
    out_ref[...] = x_ref[...] * scale


def kernel(x, batch):
    b32 = batch.astype(jnp.int32)
    bfull = jnp.concatenate([
        b32,
        jnp.full((PAD127,), B - 1, jnp.int32),
        jnp.full((B,), B, jnp.int32),
    ]).reshape(ROWS, 128)

    out = pl.pallas_call(
        _fused_kernel,
        grid=(N // BLK,),
        in_specs=[
            pl.BlockSpec((BLK, D), lambda i: (i, 0)),
            pl.BlockSpec((ROWS, 128), lambda i: (0, 0)),
        ],
        out_specs=pl.BlockSpec((BLK, D), lambda i: (i, 0)),
        out_shape=jax.ShapeDtypeStruct(x.shape, x.dtype),
        scratch_shapes=[
            pltpu.VMEM((2, B), jnp.int32),
            pltpu.VMEM((1, B), jnp.float32),
        ],
    )(x, bfull)
    return out
